# baseline (speedup 1.0000x reference)
import functools

import jax
import jax.numpy as jnp
from jax import lax
from jax.experimental import pallas as pl
from jax.experimental.pallas import tpu as pltpu
from jax.experimental.pallas import tpu_sc as plsc

NC = 2
NS = 16
NW = NC * NS
LANES = 16

N = 10000
NPAD = 10240
E = 320000
D = 128
DH = D // 2
HID = 128
C = 40

CW = 125
NCHT = E // (NS * CW)
ECHT = NCHT * CW
NCHH = NCHT // 2
NB = 4
WAVES_H = NCHH // NB
RPT = NPAD // NS
ZR = 128
HIST_EPT = E // NW


@functools.cache
def _mesh():
    return plsc.VectorSubcoreMesh(
        core_axis_name="c", subcore_axis_name="s",
        num_cores=NC, num_subcores=NS)


SLICE = NPAD // NS
HEPT = E // NS
SEPT = E // NW
XCH = 80


def _sc_pre_body(src_hbm, dst_hbm, x_hbm, invs_out, hs_out, s_out,
                 ia, ib, acc_v, invs_v, xbuf, tmp, tmp2, stage, invs_sh):
    cid = lax.axis_index("c")
    sid = lax.axis_index("s")
    wid = sid * NC + cid
    zeros = jnp.zeros((LANES,), jnp.float32)
    ones = jnp.ones((LANES,), jnp.float32)

    def zacc(i, c):
        acc_v[pl.ds(i * LANES, LANES)] = zeros
        return c

    lax.fori_loop(0, NPAD // LANES, zacc, 0)
    for h in range(2):
        pltpu.sync_copy(
            dst_hbm.at[pl.ds(sid * HEPT + h * (HEPT // 2), HEPT // 2)], ia)

        def hbody(i, c):
            idx = ia[pl.ds(i * LANES, LANES)]
            plsc.addupdate_scatter(acc_v, [idx], ones)
            return c

        lax.fori_loop(0, HEPT // 2 // LANES, hbody, 0)
    pltpu.sync_copy(acc_v, stage.at[sid])
    plsc.subcore_barrier()

    def zinv(i, c):
        invs_v[pl.ds(i * LANES, LANES)] = zeros
        return c

    lax.fori_loop(0, SLICE // LANES, zinv, 0)
    for p in range(NS):
        pltpu.sync_copy(stage.at[p].at[pl.ds(sid * SLICE, SLICE)], tmp)

        def radd(i, c):
            sl = pl.ds(i * LANES, LANES)
            invs_v[sl] = invs_v[sl] + tmp[sl]
            return c

        lax.fori_loop(0, SLICE // LANES, radd, 0)

    def rsq(i, c):
        sl = pl.ds(i * LANES, LANES)
        d = 1.0 + invs_v[sl]
        bits = plsc.bitcast(d, jnp.int32)
        y = plsc.bitcast(0x5F3759DF - lax.shift_right_logical(bits, 1),
                         jnp.float32)
        for _ in range(3):
            y = y * (1.5 - 0.5 * d * y * y)
        invs_v[sl] = y
        return c

    lax.fori_loop(0, SLICE // LANES, rsq, 0)
    pltpu.sync_copy(invs_v.at[pl.ds(0, SLICE)],
                    invs_sh.at[pl.ds(sid * SLICE, SLICE)])

    @pl.when(cid == 0)
    def _():
        pltpu.sync_copy(invs_v.at[pl.ds(0, SLICE)],
                        invs_out.at[pl.ds(sid * SLICE, SLICE)])

    plsc.subcore_barrier()
    pltpu.sync_copy(invs_sh, invs_v)

    for t in range(SLICE // XCH):
        r0 = sid * SLICE + t * XCH
        pltpu.sync_copy(x_hbm.at[cid].at[pl.ds(r0, XCH)], xbuf)

        def xrow(r, c):
            iv = plsc.load_gather(
                invs_v, [jnp.broadcast_to(r0 + r, (LANES,)).astype(jnp.int32)])
            for c4 in range(DH // LANES):
                sl = pl.ds(c4 * LANES, LANES)
                xbuf[r, sl] = xbuf[r, sl] * iv
            return c

        lax.fori_loop(0, XCH, xrow, 0)
        pltpu.sync_copy(xbuf, hs_out.at[cid].at[pl.ds(r0, XCH)])

    pltpu.sync_copy(src_hbm.at[pl.ds(wid * SEPT, SEPT)], ia)
    pltpu.sync_copy(dst_hbm.at[pl.ds(wid * SEPT, SEPT)], ib)
    lax.fori_loop(0, NPAD // LANES, zacc, 0)

    def sbody(i, c):
        d16 = ib[pl.ds(i * LANES, LANES)]
        s16 = ia[pl.ds(i * LANES, LANES)]
        vals = plsc.load_gather(invs_v, [d16])
        plsc.addupdate_scatter(acc_v, [s16], vals)
        return c

    lax.fori_loop(0, SEPT // LANES, sbody, 0)
    pltpu.sync_copy(acc_v, stage.at[sid])
    plsc.subcore_barrier()

    def ztmp(i, c):
        tmp[pl.ds(i * LANES, LANES)] = zeros
        return c

    lax.fori_loop(0, SLICE // LANES, ztmp, 0)
    for p in range(NS):
        pltpu.sync_copy(stage.at[p].at[pl.ds(sid * SLICE, SLICE)], tmp2)

        def sadd(i, c):
            sl = pl.ds(i * LANES, LANES)
            tmp[sl] = tmp[sl] + tmp2[sl]
            return c

        lax.fori_loop(0, SLICE // LANES, sadd, 0)
    pltpu.sync_copy(tmp, s_out.at[cid].at[pl.ds(sid * SLICE, SLICE)])


@functools.cache
def _sc_pre():
    return pl.kernel(
        _sc_pre_body,
        out_type=[
            jax.ShapeDtypeStruct((NPAD,), jnp.float32),
            jax.ShapeDtypeStruct((NC, NPAD, DH), jnp.float32),
            jax.ShapeDtypeStruct((NC, NPAD), jnp.float32),
        ],
        mesh=_mesh(),
        scratch_types=[
            pltpu.VMEM((HEPT // 2,), jnp.int32),
            pltpu.VMEM((SEPT,), jnp.int32),
            pltpu.VMEM((NPAD,), jnp.float32),
            pltpu.VMEM((NPAD,), jnp.float32),
            pltpu.VMEM((XCH, DH), jnp.float32),
            pltpu.VMEM((SLICE,), jnp.float32),
            pltpu.VMEM((SLICE,), jnp.float32),
            pltpu.VMEM_SHARED((NS, NPAD), jnp.float32),
            pltpu.VMEM_SHARED((NPAD,), jnp.float32),
        ],
        compiler_params=pltpu.CompilerParams(needs_layout_passes=False),
    )


def _sc_agg_body(src_hbm, dst_hbm, hs_hbm, out_hbm, *rest):
    bufs = rest[:NB]
    src_v, dst_v, zbuf, agg_sh = rest[NB:NB + 4]
    gsem = rest[NB + 4:2 * NB + 4]
    ssem = rest[2 * NB + 4:3 * NB + 4]

    cid = lax.axis_index("c")
    sid = lax.axis_index("s")

    zeros = jnp.zeros((LANES,), jnp.float32)

    def zb(i, c):
        r = i // (DH // LANES)
        col = (i % (DH // LANES)) * LANES
        zbuf[r, pl.ds(col, LANES)] = zeros
        return c

    lax.fori_loop(0, ZR * DH // LANES, zb, 0)

    for t in range(RPT // ZR):
        pltpu.async_copy(zbuf, agg_sh.at[pl.ds(sid * RPT + t * ZR, ZR)],
                         gsem[t % NB])
    for t in range(RPT // ZR):
        pltpu.make_async_copy(zbuf, agg_sh.at[pl.ds(sid * RPT + t * ZR, ZR)],
                              gsem[t % NB]).wait()
    plsc.subcore_barrier()

    hsv = hs_hbm.at[cid]

    for h in range(2):
        base = sid * NCHT + h * NCHH
        pltpu.sync_copy(src_hbm.at[pl.ds(base, NCHH)], src_v)
        pltpu.sync_copy(dst_hbm.at[pl.ds(base, NCHH)], dst_v)

        for c in range(NB):
            pltpu.async_copy(hsv.at[src_v.at[c]], bufs[c], gsem[c])

        def wave(i, carry):
            @pl.when(i > 0)
            def _():
                for c in range(NB):
                    j = i * NB + c
                    pltpu.make_async_copy(
                        bufs[c], agg_sh.at[dst_v.at[j - NB]], ssem[c]).wait()
                    pltpu.async_copy(hsv.at[src_v.at[j]], bufs[c], gsem[c])

            for c in range(NB):
                j = i * NB + c
                pltpu.make_async_copy(
                    hsv.at[src_v.at[j]], bufs[c], gsem[c]).wait()
                pltpu.async_copy(
                    bufs[c], agg_sh.at[dst_v.at[j]], ssem[c], add=True)
            return carry

        lax.fori_loop(0, WAVES_H, wave, 0)

        for c in range(NB):
            j = (WAVES_H - 1) * NB + c
            pltpu.make_async_copy(
                bufs[c], agg_sh.at[dst_v.at[j]], ssem[c]).wait()

    plsc.subcore_barrier()
    pltpu.sync_copy(agg_sh.at[pl.ds(sid * RPT, RPT)],
                    out_hbm.at[cid].at[pl.ds(sid * RPT, RPT)])


@functools.cache
def _sc_agg():
    scratch = [pltpu.VMEM((CW, DH), jnp.float32) for _ in range(NB)]
    scratch += [
        pltpu.VMEM((NCHH, CW), jnp.int32),
        pltpu.VMEM((NCHH, CW), jnp.int32),
        pltpu.VMEM((ZR, DH), jnp.float32),
        pltpu.VMEM_SHARED((NPAD, DH), jnp.float32),
    ]
    scratch += [pltpu.SemaphoreType.DMA for _ in range(2 * NB)]
    return pl.kernel(
        _sc_agg_body,
        out_type=jax.ShapeDtypeStruct((NC, NPAD, DH), jnp.float32),
        mesh=_mesh(),
        scratch_types=scratch,
        compiler_params=pltpu.CompilerParams(
            needs_layout_passes=False, use_tc_tiling_on_sc=False),
    )


_R = 1024


def _tc_agg_h(scat_ref, hs_ref, invs_ref, w_ref, b_ref):
    invs = invs_ref[...]
    agg_lo = invs * (scat_ref[0] + hs_ref[0])
    agg_hi = invs * (scat_ref[1] + hs_ref[1])
    pre = (jnp.dot(agg_lo, w_ref[:DH, :], preferred_element_type=jnp.float32)
           + jnp.dot(agg_hi, w_ref[DH:, :], preferred_element_type=jnp.float32)
           + b_ref[...][None, :])
    return jnp.maximum(pre, 0.0)


def _tc_layer_body(scat_ref, hs_ref, invs_ref, w_ref, b_ref, out_ref):
    h = _tc_agg_h(scat_ref, hs_ref, invs_ref, w_ref, b_ref)
    hsn = h * invs_ref[...]
    out_ref[0] = hsn[:, :DH]
    out_ref[1] = hsn[:, DH:]


def _tc_layer(scat, hs, invs, w, b):
    return pl.pallas_call(
        _tc_layer_body,
        grid=(NPAD // _R,),
        in_specs=[
            pl.BlockSpec((NC, _R, DH), lambda i: (0, i, 0)),
            pl.BlockSpec((NC, _R, DH), lambda i: (0, i, 0)),
            pl.BlockSpec((_R, 1), lambda i: (i, 0)),
            pl.BlockSpec((D, HID), lambda i: (0, 0)),
            pl.BlockSpec((HID,), lambda i: (0,)),
        ],
        out_specs=pl.BlockSpec((NC, _R, DH), lambda i: (0, i, 0)),
        out_shape=jax.ShapeDtypeStruct((NC, NPAD, DH), jnp.float32),
    )(scat, hs, invs, w, b)


def _tc_final_body(scat_ref, hs_ref, invs_ref, sstage_ref, mask_ref,
                   w2_ref, b2_ref, w3_ref, b3_ref, out_ref, acc_ref):
    i = pl.program_id(0)
    h3 = _tc_agg_h(scat_ref, hs_ref, invs_ref, w2_ref, b2_ref)
    invs = invs_ref[...]
    s = jnp.sum(sstage_ref[...], axis=0)[:, None]
    w = mask_ref[...] * invs * (invs + s)
    contrib = jnp.sum(w * h3, axis=0, keepdims=True)

    @pl.when(i == 0)
    def _():
        acc_ref[...] = contrib

    @pl.when(i > 0)
    def _():
        acc_ref[...] = acc_ref[...] + contrib

    @pl.when(i == NPAD // _R - 1)
    def _():
        pooled = acc_ref[...] * (1.0 / N)
        out_ref[...] = (
            jnp.dot(pooled, w3_ref[...], preferred_element_type=jnp.float32)
            + b3_ref[...][None, :])


def _tc_final(scat, hs, invs, sstage, mask, w2, b2, w3, b3):
    return pl.pallas_call(
        _tc_final_body,
        grid=(NPAD // _R,),
        in_specs=[
            pl.BlockSpec((NC, _R, DH), lambda i: (0, i, 0)),
            pl.BlockSpec((NC, _R, DH), lambda i: (0, i, 0)),
            pl.BlockSpec((_R, 1), lambda i: (i, 0)),
            pl.BlockSpec((NC, _R), lambda i: (0, i)),
            pl.BlockSpec((_R, 1), lambda i: (i, 0)),
            pl.BlockSpec((HID, HID), lambda i: (0, 0)),
            pl.BlockSpec((HID,), lambda i: (0,)),
            pl.BlockSpec((HID, C), lambda i: (0, 0)),
            pl.BlockSpec((C,), lambda i: (0,)),
        ],
        out_specs=pl.BlockSpec((1, C), lambda i: (0, 0)),
        out_shape=jax.ShapeDtypeStruct((1, C), jnp.float32),
        scratch_shapes=[pltpu.VMEM((1, HID), jnp.float32)],
    )(scat, hs, invs, sstage, mask, w2, b2, w3, b3)


def kernel(X, edge_list, W0, b0, W1, b1, W2, b2, W3, b3):
    src_flat = edge_list[0]
    dst_flat = edge_list[1]
    src2d = src_flat.reshape(NS * NCHT, CW)
    dst2d = dst_flat.reshape(NS * NCHT, CW)
    x_pad = jnp.zeros((NPAD, D), jnp.float32).at[:N].set(X)
    xsplit = jnp.stack([x_pad[:, :DH], x_pad[:, DH:]])
    mask = (jnp.arange(NPAD) < N).astype(jnp.float32)[:, None]

    invs_flat, hs, sstage = _sc_pre()(src_flat, dst_flat, xsplit)
    invs = invs_flat.reshape(NPAD, 1)

    scat0 = _sc_agg()(src2d, dst2d, hs)
    hs = _tc_layer(scat0, hs, invs, W0, b0)
    scat1 = _sc_agg()(src2d, dst2d, hs)
    hs = _tc_layer(scat1, hs, invs, W1, b1)
    scat2 = _sc_agg()(src2d, dst2d, hs)
    return _tc_final(scat2, hs, invs, sstage, mask, W2, b2, W3, b3)


# speedup vs baseline: 1.0412x; 1.0412x over previous
import functools

import jax
import jax.numpy as jnp
from jax import lax
from jax.experimental import pallas as pl
from jax.experimental.pallas import tpu as pltpu
from jax.experimental.pallas import tpu_sc as plsc

NC = 2
NS = 16
NW = NC * NS
LANES = 16

N = 10000
NPAD = 10240
E = 320000
D = 128
DH = D // 2
HID = 128
C = 40

CW = 125
NCHT = E // (NS * CW)
ECHT = NCHT * CW
NQ = 4
NCHQ = NCHT // NQ
NB = 4
WAVES_Q = NCHQ // NB
RPT = NPAD // NS
ZR = 64
HIST_EPT = E // NW


@functools.cache
def _mesh():
    return plsc.VectorSubcoreMesh(
        core_axis_name="c", subcore_axis_name="s",
        num_cores=NC, num_subcores=NS)


def _sc_hist_body(dst_hbm, out_hbm, idx_v, hist_v):
    wid = lax.axis_index("s") * NC + lax.axis_index("c")
    pltpu.sync_copy(dst_hbm.at[pl.ds(wid * HIST_EPT, HIST_EPT)], idx_v)
    zeros = jnp.zeros((LANES,), jnp.float32)

    def zbody(i, c):
        hist_v[pl.ds(i * LANES, LANES)] = zeros
        return c

    lax.fori_loop(0, NPAD // LANES, zbody, 0)
    ones = jnp.ones((LANES,), jnp.float32)

    def body(i, c):
        idx = idx_v[pl.ds(i * LANES, LANES)]
        plsc.addupdate_scatter(hist_v, [idx], ones)
        return c

    lax.fori_loop(0, HIST_EPT // LANES, body, 0)
    pltpu.sync_copy(hist_v, out_hbm.at[wid])


@functools.cache
def _sc_hist():
    return pl.kernel(
        _sc_hist_body,
        out_type=jax.ShapeDtypeStruct((NW, NPAD), jnp.float32),
        mesh=_mesh(),
        scratch_types=[
            pltpu.VMEM((HIST_EPT,), jnp.int32),
            pltpu.VMEM((NPAD,), jnp.float32),
        ],
        compiler_params=pltpu.CompilerParams(needs_layout_passes=False),
    )


def _make_agg_body(with_s):
    def body(*refs):
        if with_s:
            (src_hbm, dst_hbm, hs_hbm, invs_hbm, out_hbm, s_out, *rest) = refs
        else:
            (src_hbm, dst_hbm, hs_hbm, out_hbm, *rest) = refs
        bufs = rest[:NB]
        src_v, dst_v, zbuf, agg_sh = rest[NB:NB + 4]
        rest = rest[NB + 4:]
        if with_s:
            invs_v, vbuf, s_sh = rest[:3]
            rest = rest[3:]
        gsem = rest[:NB]
        ssem = rest[NB:2 * NB]
        if with_s:
            vsem = rest[2 * NB]

        cid = lax.axis_index("c")
        sid = lax.axis_index("s")

        zeros = jnp.zeros((LANES,), jnp.float32)

        def zb(i, c):
            r = i // (DH // LANES)
            col = (i % (DH // LANES)) * LANES
            zbuf[r, pl.ds(col, LANES)] = zeros
            return c

        lax.fori_loop(0, ZR * DH // LANES, zb, 0)
        if with_s:
            pltpu.sync_copy(invs_hbm, invs_v)

        for t in range(RPT // ZR):
            pltpu.async_copy(zbuf, agg_sh.at[pl.ds(sid * RPT + t * ZR, ZR)],
                             gsem[t % NB])
        if with_s:
            zrow = zbuf.at[0]
            for t in range(RPT // ZR):
                pltpu.async_copy(zrow, s_sh.at[pl.ds(sid * RPT + t * ZR, ZR)],
                                 ssem[t % NB])
        for t in range(RPT // ZR):
            pltpu.make_async_copy(
                zbuf, agg_sh.at[pl.ds(sid * RPT + t * ZR, ZR)],
                gsem[t % NB]).wait()
        if with_s:
            zrow = zbuf.at[0]
            for t in range(RPT // ZR):
                pltpu.make_async_copy(
                    zrow, s_sh.at[pl.ds(sid * RPT + t * ZR, ZR)],
                    ssem[t % NB]).wait()
        plsc.subcore_barrier()

        hsv = hs_hbm.at[cid]
        cols = [jnp.minimum(jnp.arange(LANES, dtype=jnp.int32) + k * LANES,
                            CW - 1) for k in range(CW // LANES + 1)]

        for q in range(NQ):
            base = sid * NCHT + q * NCHQ
            pltpu.sync_copy(src_hbm.at[pl.ds(base, NCHQ)], src_v)
            pltpu.sync_copy(dst_hbm.at[pl.ds(base, NCHQ)], dst_v)

            for c in range(NB):
                pltpu.async_copy(hsv.at[src_v.at[c]], bufs[c], gsem[c])

            def wave(i, carry):
                @pl.when(i > 0)
                def _():
                    for c in range(NB):
                        j = i * NB + c
                        pltpu.make_async_copy(
                            bufs[c], agg_sh.at[dst_v.at[j - NB]],
                            ssem[c]).wait()
                        pltpu.async_copy(hsv.at[src_v.at[j]], bufs[c],
                                         gsem[c])

                if with_s:
                    @pl.when(cid == q // 2)
                    def _():
                        for c in range(NB):
                            jj = i * NB + c
                            par = c % 2

                            @pl.when(jj >= 2)
                            def _():
                                pltpu.make_async_copy(
                                    vbuf.at[par].at[pl.ds(0, CW)],
                                    s_sh.at[src_v.at[jj - 2]], vsem).wait()

                            rsp = jnp.broadcast_to(jj, (LANES,)).astype(
                                jnp.int32)
                            for k in range(CW // LANES + 1):
                                d16 = plsc.load_gather(dst_v, [rsp, cols[k]])
                                v16 = plsc.load_gather(invs_v, [d16])
                                vbuf[par, pl.ds(k * LANES, LANES)] = v16
                            pltpu.async_copy(
                                vbuf.at[par].at[pl.ds(0, CW)],
                                s_sh.at[src_v.at[jj]], vsem, add=True)

                for c in range(NB):
                    j = i * NB + c
                    pltpu.make_async_copy(
                        hsv.at[src_v.at[j]], bufs[c], gsem[c]).wait()
                    pltpu.async_copy(
                        bufs[c], agg_sh.at[dst_v.at[j]], ssem[c], add=True)
                return carry

            lax.fori_loop(0, WAVES_Q, wave, 0)

            for c in range(NB):
                j = (WAVES_Q - 1) * NB + c
                pltpu.make_async_copy(
                    bufs[c], agg_sh.at[dst_v.at[j]], ssem[c]).wait()
            if with_s:
                @pl.when(cid == q // 2)
                def _():
                    for jj in (NCHQ - 2, NCHQ - 1):
                        pltpu.make_async_copy(
                            vbuf.at[jj % 2].at[pl.ds(0, CW)],
                            s_sh.at[src_v.at[jj]], vsem).wait()

        plsc.subcore_barrier()
        pltpu.sync_copy(agg_sh.at[pl.ds(sid * RPT, RPT)],
                        out_hbm.at[cid].at[pl.ds(sid * RPT, RPT)])
        if with_s:
            pltpu.sync_copy(s_sh.at[pl.ds(sid * RPT, RPT)],
                            s_out.at[cid].at[pl.ds(sid * RPT, RPT)])

    return body


@functools.cache
def _sc_agg(with_s):
    out_type = [jax.ShapeDtypeStruct((NC, NPAD, DH), jnp.float32)]
    if with_s:
        out_type.append(jax.ShapeDtypeStruct((NC, NPAD), jnp.float32))
    scratch = [pltpu.VMEM((CW, DH), jnp.float32) for _ in range(NB)]
    scratch += [
        pltpu.VMEM((NCHQ, CW), jnp.int32),
        pltpu.VMEM((NCHQ, CW), jnp.int32),
        pltpu.VMEM((ZR, DH), jnp.float32),
        pltpu.VMEM_SHARED((NPAD, DH), jnp.float32),
    ]
    if with_s:
        scratch += [
            pltpu.VMEM((NPAD,), jnp.float32),
            pltpu.VMEM((2, 128), jnp.float32),
            pltpu.VMEM_SHARED((NPAD,), jnp.float32),
        ]
    scratch += [pltpu.SemaphoreType.DMA for _ in range(2 * NB)]
    if with_s:
        scratch.append(pltpu.SemaphoreType.DMA)
    return pl.kernel(
        _make_agg_body(with_s),
        out_type=out_type,
        mesh=_mesh(),
        scratch_types=scratch,
        compiler_params=pltpu.CompilerParams(
            needs_layout_passes=False, use_tc_tiling_on_sc=False),
    )


_R = 1024


def _tc_prep_body(hist_ref, x_ref, invs_ref, hs_ref):
    deg = 1.0 + jnp.sum(hist_ref[...], axis=0)
    invs = lax.rsqrt(deg)
    invs_ref[...] = invs[:, None]
    hs = x_ref[...] * invs[:, None]
    hs_ref[0] = hs[:, :DH]
    hs_ref[1] = hs[:, DH:]


def _tc_prep(hist, x_pad):
    return pl.pallas_call(
        _tc_prep_body,
        grid=(NPAD // _R,),
        in_specs=[
            pl.BlockSpec((NW, _R), lambda i: (0, i)),
            pl.BlockSpec((_R, D), lambda i: (i, 0)),
        ],
        out_specs=[
            pl.BlockSpec((_R, 1), lambda i: (i, 0)),
            pl.BlockSpec((NC, _R, DH), lambda i: (0, i, 0)),
        ],
        out_shape=[
            jax.ShapeDtypeStruct((NPAD, 1), jnp.float32),
            jax.ShapeDtypeStruct((NC, NPAD, DH), jnp.float32),
        ],
    )(hist, x_pad)


def _tc_agg_h(scat_ref, hs_ref, invs_ref, w_ref, b_ref):
    invs = invs_ref[...]
    agg_lo = invs * (scat_ref[0] + hs_ref[0])
    agg_hi = invs * (scat_ref[1] + hs_ref[1])
    pre = (jnp.dot(agg_lo, w_ref[:DH, :], preferred_element_type=jnp.float32)
           + jnp.dot(agg_hi, w_ref[DH:, :], preferred_element_type=jnp.float32)
           + b_ref[...][None, :])
    return jnp.maximum(pre, 0.0)


def _tc_layer_body(scat_ref, hs_ref, invs_ref, w_ref, b_ref, out_ref):
    h = _tc_agg_h(scat_ref, hs_ref, invs_ref, w_ref, b_ref)
    hsn = h * invs_ref[...]
    out_ref[0] = hsn[:, :DH]
    out_ref[1] = hsn[:, DH:]


def _tc_layer(scat, hs, invs, w, b):
    return pl.pallas_call(
        _tc_layer_body,
        grid=(NPAD // _R,),
        in_specs=[
            pl.BlockSpec((NC, _R, DH), lambda i: (0, i, 0)),
            pl.BlockSpec((NC, _R, DH), lambda i: (0, i, 0)),
            pl.BlockSpec((_R, 1), lambda i: (i, 0)),
            pl.BlockSpec((D, HID), lambda i: (0, 0)),
            pl.BlockSpec((HID,), lambda i: (0,)),
        ],
        out_specs=pl.BlockSpec((NC, _R, DH), lambda i: (0, i, 0)),
        out_shape=jax.ShapeDtypeStruct((NC, NPAD, DH), jnp.float32),
    )(scat, hs, invs, w, b)


def _tc_final_body(scat_ref, hs_ref, invs_ref, sstage_ref, mask_ref,
                   w2_ref, b2_ref, w3_ref, b3_ref, out_ref, acc_ref):
    i = pl.program_id(0)
    h3 = _tc_agg_h(scat_ref, hs_ref, invs_ref, w2_ref, b2_ref)
    invs = invs_ref[...]
    s = jnp.sum(sstage_ref[...], axis=0)[:, None]
    w = mask_ref[...] * invs * (invs + s)
    contrib = jnp.sum(w * h3, axis=0, keepdims=True)

    @pl.when(i == 0)
    def _():
        acc_ref[...] = contrib

    @pl.when(i > 0)
    def _():
        acc_ref[...] = acc_ref[...] + contrib

    @pl.when(i == NPAD // _R - 1)
    def _():
        pooled = acc_ref[...] * (1.0 / N)
        out_ref[...] = (
            jnp.dot(pooled, w3_ref[...], preferred_element_type=jnp.float32)
            + b3_ref[...][None, :])


def _tc_final(scat, hs, invs, sstage, mask, w2, b2, w3, b3):
    return pl.pallas_call(
        _tc_final_body,
        grid=(NPAD // _R,),
        in_specs=[
            pl.BlockSpec((NC, _R, DH), lambda i: (0, i, 0)),
            pl.BlockSpec((NC, _R, DH), lambda i: (0, i, 0)),
            pl.BlockSpec((_R, 1), lambda i: (i, 0)),
            pl.BlockSpec((NC, _R), lambda i: (0, i)),
            pl.BlockSpec((_R, 1), lambda i: (i, 0)),
            pl.BlockSpec((HID, HID), lambda i: (0, 0)),
            pl.BlockSpec((HID,), lambda i: (0,)),
            pl.BlockSpec((HID, C), lambda i: (0, 0)),
            pl.BlockSpec((C,), lambda i: (0,)),
        ],
        out_specs=pl.BlockSpec((1, C), lambda i: (0, 0)),
        out_shape=jax.ShapeDtypeStruct((1, C), jnp.float32),
        scratch_shapes=[pltpu.VMEM((1, HID), jnp.float32)],
    )(scat, hs, invs, sstage, mask, w2, b2, w3, b3)


def kernel(X, edge_list, W0, b0, W1, b1, W2, b2, W3, b3):
    src_flat = edge_list[0]
    dst_flat = edge_list[1]
    src2d = src_flat.reshape(NS * NCHT, CW)
    dst2d = dst_flat.reshape(NS * NCHT, CW)
    x_pad = jnp.zeros((NPAD, D), jnp.float32).at[:N].set(X)
    mask = (jnp.arange(NPAD) < N).astype(jnp.float32)[:, None]

    hist = _sc_hist()(dst_flat)
    invs, hs = _tc_prep(hist, x_pad)

    scat0, sstage = _sc_agg(True)(src2d, dst2d, hs, invs.reshape(NPAD))
    hs = _tc_layer(scat0, hs, invs, W0, b0)
    scat1, = _sc_agg(False)(src2d, dst2d, hs)
    hs = _tc_layer(scat1, hs, invs, W1, b1)
    scat2, = _sc_agg(False)(src2d, dst2d, hs)
    return _tc_final(scat2, hs, invs, sstage, mask, W2, b2, W3, b3)


# speedup vs baseline: 1.0650x; 1.0229x over previous
import functools

import jax
import jax.numpy as jnp
from jax import lax
from jax.experimental import pallas as pl
from jax.experimental.pallas import tpu as pltpu
from jax.experimental.pallas import tpu_sc as plsc

NC = 2
NS = 16
NW = NC * NS
LANES = 16

N = 10000
NPAD = 10240
E = 320000
D = 128
DH = D // 2
HID = 128
C = 40

CW = 125
NCHT = E // (NS * CW)
ECHT = NCHT * CW
NCHH = NCHT // 2
NB = 5
WAVES_H = NCHH // NB
RPT = NPAD // NS
ZR = 128
HIST_EPT = E // NW


@functools.cache
def _mesh():
    return plsc.VectorSubcoreMesh(
        core_axis_name="c", subcore_axis_name="s",
        num_cores=NC, num_subcores=NS)


def _sc_hist_body(dst_hbm, out_hbm, idx_v, hist_v):
    wid = lax.axis_index("s") * NC + lax.axis_index("c")
    pltpu.sync_copy(dst_hbm.at[pl.ds(wid * HIST_EPT, HIST_EPT)], idx_v)
    zeros = jnp.zeros((LANES,), jnp.float32)

    def zbody(i, c):
        hist_v[pl.ds(i * LANES, LANES)] = zeros
        return c

    lax.fori_loop(0, NPAD // LANES, zbody, 0)
    ones = jnp.ones((LANES,), jnp.float32)

    def body(i, c):
        idx = idx_v[pl.ds(i * LANES, LANES)]
        plsc.addupdate_scatter(hist_v, [idx], ones)
        return c

    lax.fori_loop(0, HIST_EPT // LANES, body, 0)
    pltpu.sync_copy(hist_v, out_hbm.at[wid])


@functools.cache
def _sc_hist():
    return pl.kernel(
        _sc_hist_body,
        out_type=jax.ShapeDtypeStruct((NW, NPAD), jnp.float32),
        mesh=_mesh(),
        scratch_types=[
            pltpu.VMEM((HIST_EPT,), jnp.int32),
            pltpu.VMEM((NPAD,), jnp.float32),
        ],
        compiler_params=pltpu.CompilerParams(needs_layout_passes=False),
    )


def _sc_s_body(src_hbm, dst_hbm, invs_hbm, out_hbm, src_v, dst_v, invs_v, s_v):
    wid = lax.axis_index("s") * NC + lax.axis_index("c")
    pltpu.sync_copy(src_hbm.at[pl.ds(wid * HIST_EPT, HIST_EPT)], src_v)
    pltpu.sync_copy(dst_hbm.at[pl.ds(wid * HIST_EPT, HIST_EPT)], dst_v)
    pltpu.sync_copy(invs_hbm, invs_v)
    zeros = jnp.zeros((LANES,), jnp.float32)

    def zbody(i, c):
        s_v[pl.ds(i * LANES, LANES)] = zeros
        return c

    lax.fori_loop(0, NPAD // LANES, zbody, 0)

    def body(i, c):
        d16 = dst_v[pl.ds(i * LANES, LANES)]
        s16 = src_v[pl.ds(i * LANES, LANES)]
        vals = plsc.load_gather(invs_v, [d16])
        plsc.addupdate_scatter(s_v, [s16], vals)
        return c

    lax.fori_loop(0, HIST_EPT // LANES, body, 0)
    pltpu.sync_copy(s_v, out_hbm.at[wid])


@functools.cache
def _sc_s():
    return pl.kernel(
        _sc_s_body,
        out_type=jax.ShapeDtypeStruct((NW, NPAD), jnp.float32),
        mesh=_mesh(),
        scratch_types=[
            pltpu.VMEM((HIST_EPT,), jnp.int32),
            pltpu.VMEM((HIST_EPT,), jnp.int32),
            pltpu.VMEM((NPAD,), jnp.float32),
            pltpu.VMEM((NPAD,), jnp.float32),
        ],
        compiler_params=pltpu.CompilerParams(needs_layout_passes=False),
    )


def _sc_agg_body(src_hbm, dst_hbm, hs_hbm, out_hbm, *rest):
    bufs = rest[:NB]
    src_v, dst_v, zbuf, agg_sh = rest[NB:NB + 4]
    gsem = rest[NB + 4:2 * NB + 4]
    ssem = rest[2 * NB + 4:3 * NB + 4]

    cid = lax.axis_index("c")
    sid = lax.axis_index("s")

    zeros = jnp.zeros((LANES,), jnp.float32)

    def zb(i, c):
        r = i // (DH // LANES)
        col = (i % (DH // LANES)) * LANES
        zbuf[r, pl.ds(col, LANES)] = zeros
        return c

    lax.fori_loop(0, ZR * DH // LANES, zb, 0)

    for t in range(RPT // ZR):
        pltpu.async_copy(zbuf, agg_sh.at[pl.ds(sid * RPT + t * ZR, ZR)],
                         gsem[t % NB])
    for t in range(RPT // ZR):
        pltpu.make_async_copy(zbuf, agg_sh.at[pl.ds(sid * RPT + t * ZR, ZR)],
                              gsem[t % NB]).wait()
    plsc.subcore_barrier()

    hsv = hs_hbm.at[cid]

    for h in range(2):
        base = sid * NCHT + h * NCHH
        pltpu.sync_copy(src_hbm.at[pl.ds(base, NCHH)], src_v)
        pltpu.sync_copy(dst_hbm.at[pl.ds(base, NCHH)], dst_v)

        for c in range(NB):
            pltpu.async_copy(hsv.at[src_v.at[c]], bufs[c], gsem[c])

        def wave(i, carry):
            @pl.when(i > 0)
            def _():
                for c in range(NB):
                    j = i * NB + c
                    pltpu.make_async_copy(
                        bufs[c], agg_sh.at[dst_v.at[j - NB]], ssem[c]).wait()
                    pltpu.async_copy(hsv.at[src_v.at[j]], bufs[c], gsem[c])

            for c in range(NB):
                j = i * NB + c
                pltpu.make_async_copy(
                    hsv.at[src_v.at[j]], bufs[c], gsem[c]).wait()
                pltpu.async_copy(
                    bufs[c], agg_sh.at[dst_v.at[j]], ssem[c], add=True)
            return carry

        lax.fori_loop(0, WAVES_H, wave, 0)

        for c in range(NB):
            j = (WAVES_H - 1) * NB + c
            pltpu.make_async_copy(
                bufs[c], agg_sh.at[dst_v.at[j]], ssem[c]).wait()

    plsc.subcore_barrier()
    pltpu.sync_copy(agg_sh.at[pl.ds(sid * RPT, RPT)],
                    out_hbm.at[cid].at[pl.ds(sid * RPT, RPT)])


@functools.cache
def _sc_agg():
    scratch = [pltpu.VMEM((CW, DH), jnp.float32) for _ in range(NB)]
    scratch += [
        pltpu.VMEM((NCHH, CW), jnp.int32),
        pltpu.VMEM((NCHH, CW), jnp.int32),
        pltpu.VMEM((ZR, DH), jnp.float32),
        pltpu.VMEM_SHARED((NPAD, DH), jnp.float32),
    ]
    scratch += [pltpu.SemaphoreType.DMA for _ in range(2 * NB)]
    return pl.kernel(
        _sc_agg_body,
        out_type=jax.ShapeDtypeStruct((NC, NPAD, DH), jnp.float32),
        mesh=_mesh(),
        scratch_types=scratch,
        compiler_params=pltpu.CompilerParams(
            needs_layout_passes=False, use_tc_tiling_on_sc=False),
    )


_R = 1024


def _tc_prep_body(hist_ref, x_ref, invs_ref, hs_ref):
    deg = 1.0 + jnp.sum(hist_ref[...], axis=0)
    invs = lax.rsqrt(deg)
    invs_ref[...] = invs[:, None]
    hs = x_ref[...] * invs[:, None]
    hs_ref[0] = hs[:, :DH]
    hs_ref[1] = hs[:, DH:]


def _tc_prep(hist, x_pad):
    return pl.pallas_call(
        _tc_prep_body,
        grid=(NPAD // _R,),
        in_specs=[
            pl.BlockSpec((NW, _R), lambda i: (0, i)),
            pl.BlockSpec((_R, D), lambda i: (i, 0)),
        ],
        out_specs=[
            pl.BlockSpec((_R, 1), lambda i: (i, 0)),
            pl.BlockSpec((NC, _R, DH), lambda i: (0, i, 0)),
        ],
        out_shape=[
            jax.ShapeDtypeStruct((NPAD, 1), jnp.float32),
            jax.ShapeDtypeStruct((NC, NPAD, DH), jnp.float32),
        ],
    )(hist, x_pad)


def _tc_agg_h(scat_ref, hs_ref, invs_ref, w_ref, b_ref):
    invs = invs_ref[...]
    agg_lo = invs * (scat_ref[0] + hs_ref[0])
    agg_hi = invs * (scat_ref[1] + hs_ref[1])
    pre = (jnp.dot(agg_lo, w_ref[:DH, :], preferred_element_type=jnp.float32)
           + jnp.dot(agg_hi, w_ref[DH:, :], preferred_element_type=jnp.float32)
           + b_ref[...][None, :])
    return jnp.maximum(pre, 0.0)


def _tc_layer_body(scat_ref, hs_ref, invs_ref, w_ref, b_ref, out_ref):
    h = _tc_agg_h(scat_ref, hs_ref, invs_ref, w_ref, b_ref)
    hsn = h * invs_ref[...]
    out_ref[0] = hsn[:, :DH]
    out_ref[1] = hsn[:, DH:]


def _tc_layer(scat, hs, invs, w, b):
    return pl.pallas_call(
        _tc_layer_body,
        grid=(NPAD // _R,),
        in_specs=[
            pl.BlockSpec((NC, _R, DH), lambda i: (0, i, 0)),
            pl.BlockSpec((NC, _R, DH), lambda i: (0, i, 0)),
            pl.BlockSpec((_R, 1), lambda i: (i, 0)),
            pl.BlockSpec((D, HID), lambda i: (0, 0)),
            pl.BlockSpec((HID,), lambda i: (0,)),
        ],
        out_specs=pl.BlockSpec((NC, _R, DH), lambda i: (0, i, 0)),
        out_shape=jax.ShapeDtypeStruct((NC, NPAD, DH), jnp.float32),
    )(scat, hs, invs, w, b)


def _tc_final_body(scat_ref, hs_ref, invs_ref, sstage_ref, mask_ref,
                   w2_ref, b2_ref, w3_ref, b3_ref, out_ref, acc_ref):
    i = pl.program_id(0)
    h3 = _tc_agg_h(scat_ref, hs_ref, invs_ref, w2_ref, b2_ref)
    invs = invs_ref[...]
    s = jnp.sum(sstage_ref[...], axis=0)[:, None]
    w = mask_ref[...] * invs * (invs + s)
    contrib = jnp.sum(w * h3, axis=0, keepdims=True)

    @pl.when(i == 0)
    def _():
        acc_ref[...] = contrib

    @pl.when(i > 0)
    def _():
        acc_ref[...] = acc_ref[...] + contrib

    @pl.when(i == NPAD // _R - 1)
    def _():
        pooled = acc_ref[...] * (1.0 / N)
        out_ref[...] = (
            jnp.dot(pooled, w3_ref[...], preferred_element_type=jnp.float32)
            + b3_ref[...][None, :])


def _tc_final(scat, hs, invs, sstage, mask, w2, b2, w3, b3):
    return pl.pallas_call(
        _tc_final_body,
        grid=(NPAD // _R,),
        in_specs=[
            pl.BlockSpec((NC, _R, DH), lambda i: (0, i, 0)),
            pl.BlockSpec((NC, _R, DH), lambda i: (0, i, 0)),
            pl.BlockSpec((_R, 1), lambda i: (i, 0)),
            pl.BlockSpec((NW, _R), lambda i: (0, i)),
            pl.BlockSpec((_R, 1), lambda i: (i, 0)),
            pl.BlockSpec((HID, HID), lambda i: (0, 0)),
            pl.BlockSpec((HID,), lambda i: (0,)),
            pl.BlockSpec((HID, C), lambda i: (0, 0)),
            pl.BlockSpec((C,), lambda i: (0,)),
        ],
        out_specs=pl.BlockSpec((1, C), lambda i: (0, 0)),
        out_shape=jax.ShapeDtypeStruct((1, C), jnp.float32),
        scratch_shapes=[pltpu.VMEM((1, HID), jnp.float32)],
    )(scat, hs, invs, sstage, mask, w2, b2, w3, b3)


def kernel(X, edge_list, W0, b0, W1, b1, W2, b2, W3, b3):
    src_flat = edge_list[0]
    dst_flat = edge_list[1]
    src2d = src_flat.reshape(NS * NCHT, CW)
    dst2d = dst_flat.reshape(NS * NCHT, CW)
    x_pad = jnp.zeros((NPAD, D), jnp.float32).at[:N].set(X)
    mask = (jnp.arange(NPAD) < N).astype(jnp.float32)[:, None]

    hist = _sc_hist()(dst_flat)
    invs, hs = _tc_prep(hist, x_pad)

    scat0 = _sc_agg()(src2d, dst2d, hs)
    sstage = _sc_s()(src_flat, dst_flat, invs.reshape(NPAD))
    hs = _tc_layer(scat0, hs, invs, W0, b0)
    scat1 = _sc_agg()(src2d, dst2d, hs)
    hs = _tc_layer(scat1, hs, invs, W1, b1)
    scat2 = _sc_agg()(src2d, dst2d, hs)
    return _tc_final(scat2, hs, invs, sstage, mask, W2, b2, W3, b3)
